# Initial kernel scaffold; baseline (speedup 1.0000x reference)
import jax
import jax.numpy as jnp
from jax.experimental import pallas as pl


def kernel(encoder_out, durations, max_len):
    raise NotImplementedError("write your pallas kernel here")


# speedup vs baseline: 15.7777x; 15.7777x over previous
import jax
import jax.numpy as jnp
from jax import lax
from jax.experimental import pallas as pl
from jax.experimental.pallas import tpu as pltpu
from jax.experimental.pallas import tpu_sc as plsc

_B = 16
_T = 512
_D = 384
_L = 2048
_NC = 2
_NS = 16
_NW = _NC * _NS
_FRAMES_PER_TILE = _B * _L // _NW
_CHUNK = 128
_NCHUNK = _FRAMES_PER_TILE // _CHUNK
_ZERO_ROW = _B * _T
_LANES = 16


def _lr_body(table_hbm, dur_hbm, ml_hbm, out_hbm, mel_hbm,
             cs_ref, dur_ref, idx_ref, ml_ref, mel_ref, rows0, rows1,
             sem0, sem1, osem):
    wid = lax.axis_index("s") * _NC + lax.axis_index("c")
    b = wid // 2
    j0 = (wid % 2) * _FRAMES_PER_TILE

    pltpu.sync_copy(dur_hbm.at[b], dur_ref)
    pltpu.sync_copy(ml_hbm, ml_ref)

    carry = jnp.int32(0)
    for i in range(_T // _LANES):
        v = dur_ref[pl.ds(i * _LANES, _LANES)]
        cs_ref[pl.ds(i * _LANES, _LANES)] = plsc.cumsum(v) + carry
        carry = carry + jnp.sum(v)
    mel_len = carry

    @pl.when(wid % 2 == 0)
    def _():
        mel_ref[...] = jnp.full((_LANES,), mel_len, jnp.int32)
        pltpu.sync_copy(mel_ref, mel_hbm.at[b])

    valid_len = jnp.minimum(jnp.full((_LANES,), mel_len, jnp.int32), ml_ref[...])
    lanes = lax.iota(jnp.int32, _LANES)

    def compute_chunk(k):
        for i in range(_CHUNK // _LANES):
            p = lanes + (j0 + k * _CHUNK + i * _LANES)
            lo = jnp.zeros((_LANES,), jnp.int32)
            hi = jnp.full((_LANES,), _T, jnp.int32)
            for _ in range(10):
                mid = (lo + hi) >> 1
                cm = plsc.load_gather(cs_ref, [mid])
                take = cm <= p
                lo = jnp.where(take, mid + 1, lo)
                hi = jnp.where(take, hi, mid)
            flat = jnp.where(p < valid_len, lo + b * _T, _ZERO_ROW)
            idx_ref[k, pl.ds(i * _LANES, _LANES)] = flat

    bufs = (rows0, rows1)
    sems = (sem0, sem1)
    compute_chunk(0)
    copies = {0: pltpu.async_copy(table_hbm.at[idx_ref.at[0]], bufs[0], sems[0])}
    for k in range(_NCHUNK):
        if k + 1 < _NCHUNK:
            compute_chunk(k + 1)
            copies[k + 1] = pltpu.async_copy(
                table_hbm.at[idx_ref.at[k + 1]], bufs[(k + 1) % 2],
                sems[(k + 1) % 2])
        copies[k].wait()
        pltpu.async_copy(
            bufs[k % 2],
            out_hbm.at[pl.ds(wid * _FRAMES_PER_TILE + k * _CHUNK, _CHUNK)],
            osem).wait()


def _sc_expand(table, durations, ml_vec):
    mesh = plsc.VectorSubcoreMesh(
        core_axis_name="c", subcore_axis_name="s",
        num_cores=_NC, num_subcores=_NS)
    fn = pl.kernel(
        _lr_body,
        out_type=(
            jax.ShapeDtypeStruct((_B * _L, _D), jnp.float32),
            jax.ShapeDtypeStruct((_B, _LANES), jnp.int32),
        ),
        mesh=mesh,
        compiler_params=pltpu.CompilerParams(needs_layout_passes=False),
        scratch_types=[
            pltpu.VMEM((_T,), jnp.int32),
            pltpu.VMEM((_T,), jnp.int32),
            pltpu.VMEM((_NCHUNK, _CHUNK), jnp.int32),
            pltpu.VMEM((_LANES,), jnp.int32),
            pltpu.VMEM((_LANES,), jnp.int32),
            pltpu.VMEM((_CHUNK, _D), jnp.float32),
            pltpu.VMEM((_CHUNK, _D), jnp.float32),
            pltpu.SemaphoreType.DMA,
            pltpu.SemaphoreType.DMA,
            pltpu.SemaphoreType.DMA,
        ],
    )
    return fn(table, durations, ml_vec)


def kernel(encoder_out, durations, max_len):
    B, T, D = encoder_out.shape
    table = jnp.concatenate(
        [encoder_out.reshape(B * T, D),
         jnp.zeros((8, D), encoder_out.dtype)], axis=0)
    ml_vec = jnp.full((_LANES,), max_len, jnp.int32)
    out_flat, mel_mat = _sc_expand(table, durations, ml_vec)
    return out_flat.reshape(B, _L, D), mel_mat[:, 0]
